# baseline (speedup 1.0000x reference)
import functools

import jax
import jax.numpy as jnp
from jax import lax
from jax.experimental import pallas as pl
from jax.experimental.pallas import tpu as pltpu
from jax.experimental.pallas import tpu_sc as plsc

N = 100000
E = 3200000
F = 16
NTILES = 16
N2 = 100096
RPT = N2 // NTILES
ZROWS = 368
BIG = 1024
NCHUNK = 198
BIG2 = 3072
NCHUNK2 = 66
EP = NTILES * BIG * NCHUNK
EPS = 1e-16

_mesh = plsc.VectorSubcoreMesh(core_axis_name="c", subcore_axis_name="s")


def _zero_1d(ref, n):
  z = jnp.zeros((16,), jnp.float32)
  def body(i, _):
    ref[pl.ds(i * 16, 16)] = z
    return 0
  lax.fori_loop(0, n // 16, body, 0)


@functools.partial(
    pl.kernel,
    out_type=[
        jax.ShapeDtypeStruct((N2, F), jnp.float32),
        jax.ShapeDtypeStruct((N2,), jnp.float32),
        jax.ShapeDtypeStruct((N2, F), jnp.float32),
        jax.ShapeDtypeStruct((N2,), jnp.float32),
    ],
    mesh=_mesh,
    compiler_params=pltpu.CompilerParams(use_tc_tiling_on_sc=False),
    scratch_types=[
        pltpu.VMEM((BIG,), jnp.int32),
        pltpu.VMEM((BIG,), jnp.int32),
        pltpu.VMEM((BIG,), jnp.float32),
        pltpu.VMEM((BIG,), jnp.float32),
        pltpu.VMEM((BIG,), jnp.float32),
        pltpu.VMEM((BIG, F), jnp.float32),
        pltpu.VMEM_SHARED((N2, F), jnp.float32),
        pltpu.VMEM_SHARED((N2,), jnp.float32),
        pltpu.SemaphoreType.DMA,
        pltpu.SemaphoreType.DMA,
        pltpu.SemaphoreType.DMA,
    ],
)
def _sc_layer1(src_h, dst_h, aFs_h, aFd_h, hF_h, aRs_h, aRd_h, hR_h,
               uF_o, denF_o, uR_o, denR_o,
               si, di, as1, ad1, wv1, h3, u_sh, den_sh,
               isem, gsem, ssem):
  cid = lax.axis_index("c")
  sid = lax.axis_index("s")
  r0 = pl.multiple_of(sid * RPT, 8)
  woffs = [min(k * BIG, RPT - BIG) for k in range(7)]

  def zrow(r, _):
    h3[r, :] = jnp.zeros((F,), jnp.float32)
    return 0
  lax.fori_loop(0, BIG, zrow, 0)
  def zv(i, _):
    wv1[pl.ds(i * 16, 16)] = jnp.zeros((16,), jnp.float32)
    return 0
  lax.fori_loop(0, BIG // 16, zv, 0)
  zds = []
  for off in woffs:
    zds.append(pltpu.async_copy(h3, u_sh.at[pl.ds(r0 + off, BIG)], ssem))
    zds.append(pltpu.async_copy(wv1, den_sh.at[pl.ds(r0 + off, BIG)], ssem))
  for d in zds:
    d.wait()
  plsc.subcore_barrier()

  e0 = sid * (BIG * NCHUNK)

  def process(s_i, d_i, aS_t, aD_t, h_t):
    descs = [pltpu.async_copy(aS_t.at[s_i], as1, gsem),
             pltpu.async_copy(aD_t.at[d_i], ad1, gsem),
             pltpu.async_copy(h_t.at[s_i], h3, gsem)]
    for d in descs:
      d.wait()

    def grp(g, _):
      o = pl.multiple_of(g * 16, 16)
      v = as1[pl.ds(o, 16)] + ad1[pl.ds(o, 16)]
      e = jnp.where(v >= 0.0, v, 0.2 * v)
      w = jnp.exp(e)
      wv1[pl.ds(o, 16)] = w
      for l in range(16):
        h3[o + l, :] = h3[o + l, :] * w[l]
      return 0
    lax.fori_loop(0, BIG // 16, grp, 0)

    sd = [pltpu.async_copy(h3, u_sh.at[d_i], ssem, add=True),
          pltpu.async_copy(wv1, den_sh.at[d_i], ssem, add=True)]
    for d in sd:
      d.wait()

  def chunk(c, _):
    base = e0 + c * BIG
    d1 = pltpu.async_copy(src_h.at[pl.ds(base, BIG)], si, isem)
    d2_ = pltpu.async_copy(dst_h.at[pl.ds(base, BIG)], di, isem)
    d1.wait()
    d2_.wait()

    @pl.when(cid == 0)
    def _():
      process(si, di, aFs_h, aFd_h, hF_h)

    @pl.when(cid == 1)
    def _():
      process(di, si, aRs_h, aRd_h, hR_h)

    return 0

  lax.fori_loop(0, NCHUNK, chunk, 0)
  plsc.subcore_barrier()

  def flush(u_o, den_o):
    for off in woffs:
      ds_ = [pltpu.async_copy(u_sh.at[pl.ds(r0 + off, BIG)], h3, gsem),
             pltpu.async_copy(den_sh.at[pl.ds(r0 + off, BIG)], wv1, gsem)]
      for d in ds_:
        d.wait()
      ds_ = [pltpu.async_copy(h3, u_o.at[pl.ds(r0 + off, BIG)], ssem),
             pltpu.async_copy(wv1, den_o.at[pl.ds(r0 + off, BIG)], ssem)]
      for d in ds_:
        d.wait()

  @pl.when(cid == 0)
  def _():
    flush(uF_o, denF_o)

  @pl.when(cid == 1)
  def _():
    flush(uR_o, denR_o)


@functools.partial(
    pl.kernel,
    out_type=[
        jax.ShapeDtypeStruct((N2,), jnp.float32),
        jax.ShapeDtypeStruct((N2,), jnp.float32),
        jax.ShapeDtypeStruct((N2,), jnp.float32),
        jax.ShapeDtypeStruct((N2,), jnp.float32),
    ],
    mesh=_mesh,
    compiler_params=pltpu.CompilerParams(use_tc_tiling_on_sc=False),
    scratch_types=[
        pltpu.VMEM((BIG2,), jnp.int32),
        pltpu.VMEM((BIG2,), jnp.int32),
        pltpu.VMEM((BIG2,), jnp.float32),
        pltpu.VMEM((BIG2,), jnp.float32),
        pltpu.VMEM((BIG2,), jnp.float32),
        pltpu.VMEM((BIG2,), jnp.float32),
        pltpu.VMEM((16,), jnp.float32),
        pltpu.SemaphoreType.DMA,
        pltpu.SemaphoreType.DMA,
        pltpu.SemaphoreType.DMA,
        pltpu.VMEM_SHARED((N2,), jnp.float32),
        pltpu.VMEM_SHARED((N2,), jnp.float32),
    ],
)
def _sc_layer2(src_h, dst_h, TF_h, TR_h, sc_h,
               u2F_o, d2F_o, u2R_o, d2R_o,
               si, di, ts, td, wv1, mv1, scv, isem, gsem, ssem,
               u_sh, den_sh):
  cid = lax.axis_index("c")
  sid = lax.axis_index("s")
  r0 = pl.multiple_of(sid * RPT, 8)
  woffs = [min(k * BIG2, RPT - BIG2) for k in range(3)]

  pltpu.sync_copy(sc_h, scv)
  scs = scv[...]
  sa = jnp.where(cid == 0, scs[0], scs[2])
  sb = jnp.where(cid == 0, scs[1], scs[3])

  def zv(i, _):
    wv1[pl.ds(i * 16, 16)] = jnp.zeros((16,), jnp.float32)
    return 0
  lax.fori_loop(0, BIG2 // 16, zv, 0)
  zds = []
  for off in woffs:
    zds.append(pltpu.async_copy(wv1, u_sh.at[pl.ds(r0 + off, BIG2)], ssem))
    zds.append(pltpu.async_copy(wv1, den_sh.at[pl.ds(r0 + off, BIG2)], ssem))
  for d in zds:
    d.wait()
  plsc.subcore_barrier()

  e0 = sid * (BIG2 * NCHUNK2)

  def process(s_i, d_i, T_t):
    descs = [pltpu.async_copy(T_t.at[s_i], ts, gsem),
             pltpu.async_copy(T_t.at[d_i], td, gsem)]
    for d in descs:
      d.wait()

    def grp(g, _):
      o = pl.multiple_of(g * 16, 16)
      vs = ts[pl.ds(o, 16)]
      v = sa * vs + sb * td[pl.ds(o, 16)]
      e = jnp.where(v >= 0.0, v, 0.2 * v)
      w = jnp.exp(e)
      wv1[pl.ds(o, 16)] = w
      mv1[pl.ds(o, 16)] = w * vs
      return 0
    lax.fori_loop(0, BIG2 // 16, grp, 0)

    sd = [pltpu.async_copy(mv1, u_sh.at[d_i], ssem, add=True),
          pltpu.async_copy(wv1, den_sh.at[d_i], ssem, add=True)]
    for d in sd:
      d.wait()

  def chunk(c, _):
    base = e0 + c * BIG2
    d1 = pltpu.async_copy(src_h.at[pl.ds(base, BIG2)], si, isem)
    d2_ = pltpu.async_copy(dst_h.at[pl.ds(base, BIG2)], di, isem)
    d1.wait()
    d2_.wait()

    @pl.when(cid == 0)
    def _():
      process(si, di, TF_h)

    @pl.when(cid == 1)
    def _():
      process(di, si, TR_h)

    return 0

  lax.fori_loop(0, NCHUNK2, chunk, 0)
  plsc.subcore_barrier()

  def flush(u_o, den_o):
    for off in woffs:
      ds_ = [pltpu.async_copy(u_sh.at[pl.ds(r0 + off, BIG2)], mv1, gsem),
             pltpu.async_copy(den_sh.at[pl.ds(r0 + off, BIG2)], wv1, gsem)]
      for d in ds_:
        d.wait()
      ds_ = [pltpu.async_copy(mv1, u_o.at[pl.ds(r0 + off, BIG2)], ssem),
             pltpu.async_copy(wv1, den_o.at[pl.ds(r0 + off, BIG2)], ssem)]
      for d in ds_:
        d.wait()

  @pl.when(cid == 0)
  def _():
    flush(u2F_o, d2F_o)

  @pl.when(cid == 1)
  def _():
    flush(u2R_o, d2R_o)


BLK = 3128


def _tca_body(x_ref, MF_ref, vasF_ref, vadF_ref, MR_ref, vasR_ref, vadR_ref,
              hF_ref, aFs_ref, aFd_ref, hR_ref, aRs_ref, aRd_ref):
  x = x_ref[...]
  hF = jnp.dot(x, MF_ref[...], preferred_element_type=jnp.float32)
  hF_ref[...] = hF
  aFs_ref[...] = jnp.dot(hF, vasF_ref[...], preferred_element_type=jnp.float32)
  aFd_ref[...] = jnp.dot(hF, vadF_ref[...], preferred_element_type=jnp.float32)
  hR = jnp.dot(x, MR_ref[...], preferred_element_type=jnp.float32)
  hR_ref[...] = hR
  aRs_ref[...] = jnp.dot(hR, vasR_ref[...], preferred_element_type=jnp.float32)
  aRd_ref[...] = jnp.dot(hR, vadR_ref[...], preferred_element_type=jnp.float32)


def _tcb_body(uF_ref, dF_ref, uR_ref, dR_ref, b1_ref, w2F_ref, b1r_ref,
              w2R_ref, TF_ref, TR_ref):
  x1F = jnp.maximum(uF_ref[...] / (dF_ref[...] + EPS) + b1_ref[...], 0.0)
  TF_ref[...] = jnp.dot(x1F, w2F_ref[...], preferred_element_type=jnp.float32)
  x1R = jnp.maximum(uR_ref[...] / (dR_ref[...] + EPS) + b1r_ref[...], 0.0)
  TR_ref[...] = jnp.dot(x1R, w2R_ref[...], preferred_element_type=jnp.float32)


def _tcc_body(u2F_ref, d2F_ref, u2R_ref, d2R_ref, bb_ref, out_ref):
  bb = bb_ref[...]
  oF = u2F_ref[...] / (d2F_ref[...] + EPS) + bb[0, 0]
  oR = u2R_ref[...] / (d2R_ref[...] + EPS) + bb[0, 1]
  out_ref[...] = (oF + oR) * 0.5


def _row_spec(cols):
  return pl.BlockSpec((BLK, cols), lambda i: (i, 0))


def _full_spec(shape):
  return pl.BlockSpec(shape, lambda i: tuple(0 for _ in shape))


def kernel(x, edge_index, W1, a1_src, a1_dst, b1, W2, a2_src, a2_dst, b2,
           W1r, a1r_src, a1r_dst, b1r, W2r, a2r_src, a2r_dst, b2r):
  pad_e = jnp.full((EP - E,), N, jnp.int32)
  src1 = jnp.concatenate([edge_index[0], pad_e])
  dst1 = jnp.concatenate([edge_index[1], pad_e])
  xp = jnp.pad(x, ((0, N2 - N), (0, 0)))

  MF = W1.T
  vasF = a1_src[:, None]
  vadF = a1_dst[:, None]
  MR = W1r.T
  vasR = a1r_src[:, None]
  vadR = a1r_dst[:, None]
  w2F = W2.T
  w2R = W2r.T
  sc16 = jnp.zeros((16,), jnp.float32).at[0].set(a2_src[0]).at[1].set(
      a2_dst[0]).at[2].set(a2r_src[0]).at[3].set(a2r_dst[0])
  bb2 = jnp.stack([b2[0], b2r[0]])[None, :]

  grid = (N2 // BLK,)
  f32 = jnp.float32

  hF, aFs, aFd, hR, aRs, aRd = pl.pallas_call(
      _tca_body,
      grid=grid,
      in_specs=[_row_spec(3), _full_spec((3, F)), _full_spec((F, 1)),
                _full_spec((F, 1)), _full_spec((3, F)), _full_spec((F, 1)),
                _full_spec((F, 1))],
      out_specs=[_row_spec(F), _row_spec(1), _row_spec(1),
                 _row_spec(F), _row_spec(1), _row_spec(1)],
      out_shape=[jax.ShapeDtypeStruct((N2, F), f32),
                 jax.ShapeDtypeStruct((N2, 1), f32),
                 jax.ShapeDtypeStruct((N2, 1), f32),
                 jax.ShapeDtypeStruct((N2, F), f32),
                 jax.ShapeDtypeStruct((N2, 1), f32),
                 jax.ShapeDtypeStruct((N2, 1), f32)],
  )(xp, MF, vasF, vadF, MR, vasR, vadR)

  uF, denF, uR, denR = _sc_layer1(
      src1, dst1, aFs.reshape(N2), aFd.reshape(N2), hF,
      aRs.reshape(N2), aRd.reshape(N2), hR)

  TF, TR = pl.pallas_call(
      _tcb_body,
      grid=grid,
      in_specs=[_row_spec(F), _row_spec(1), _row_spec(F), _row_spec(1),
                _full_spec((1, F)), _full_spec((F, 1)), _full_spec((1, F)),
                _full_spec((F, 1))],
      out_specs=[_row_spec(1)] * 2,
      out_shape=[jax.ShapeDtypeStruct((N2, 1), f32)] * 2,
  )(uF, denF.reshape(N2, 1), uR, denR.reshape(N2, 1),
    b1[None, :], w2F, b1r[None, :], w2R)

  u2F, d2F, u2R, d2R = _sc_layer2(
      src1, dst1, TF.reshape(N2), TR.reshape(N2), sc16)

  out = pl.pallas_call(
      _tcc_body,
      grid=grid,
      in_specs=[_row_spec(1), _row_spec(1), _row_spec(1), _row_spec(1),
                _full_spec((1, 2))],
      out_specs=_row_spec(1),
      out_shape=jax.ShapeDtypeStruct((N2, 1), f32),
  )(u2F.reshape(N2, 1), d2F.reshape(N2, 1), u2R.reshape(N2, 1),
    d2R.reshape(N2, 1), bb2)

  return out[:N]


# speedup vs baseline: 1.3558x; 1.3558x over previous
import functools

import jax
import jax.numpy as jnp
from jax import lax
from jax.experimental import pallas as pl
from jax.experimental.pallas import tpu as pltpu
from jax.experimental.pallas import tpu_sc as plsc

N = 100000
E = 3200000
F = 16
NTILES = 16
N2 = 100096
RPT = N2 // NTILES
ZROWS = 368
BIG = 512
NPAIR = 198
BIG2 = 3072
NPAIR2 = 33
EP = NTILES * BIG * 2 * NPAIR
EPS = 1e-16

_mesh = plsc.VectorSubcoreMesh(core_axis_name="c", subcore_axis_name="s")


def _zero_1d(ref, n):
  z = jnp.zeros((16,), jnp.float32)
  def body(i, _):
    ref[pl.ds(i * 16, 16)] = z
    return 0
  lax.fori_loop(0, n // 16, body, 0)


@functools.partial(
    pl.kernel,
    out_type=[
        jax.ShapeDtypeStruct((N2, F), jnp.float32),
        jax.ShapeDtypeStruct((N2,), jnp.float32),
        jax.ShapeDtypeStruct((N2, F), jnp.float32),
        jax.ShapeDtypeStruct((N2,), jnp.float32),
    ],
    mesh=_mesh,
    compiler_params=pltpu.CompilerParams(use_tc_tiling_on_sc=False),
    scratch_types=[
        pltpu.VMEM((2, BIG), jnp.int32),
        pltpu.VMEM((2, BIG), jnp.int32),
        pltpu.VMEM((2, BIG), jnp.float32),
        pltpu.VMEM((2, BIG), jnp.float32),
        pltpu.VMEM((2, BIG), jnp.float32),
        pltpu.VMEM((2, BIG, F), jnp.float32),
        pltpu.VMEM_SHARED((N2, F), jnp.float32),
        pltpu.VMEM_SHARED((N2,), jnp.float32),
        pltpu.SemaphoreType.DMA,
        pltpu.SemaphoreType.DMA,
        pltpu.SemaphoreType.DMA,
        pltpu.SemaphoreType.DMA,
        pltpu.SemaphoreType.DMA,
        pltpu.SemaphoreType.DMA,
    ],
)
def _sc_layer1(src_h, dst_h, aFs_h, aFd_h, hF_h, aRs_h, aRd_h, hR_h,
               uF_o, denF_o, uR_o, denR_o,
               si, di, as1, ad1, wv1, h3, u_sh, den_sh,
               isem0, isem1, gsem0, gsem1, ssem0, ssem1):
  cid = lax.axis_index("c")
  sid = lax.axis_index("s")
  r0 = pl.multiple_of(sid * RPT, 8)
  isems = [isem0, isem1]
  gsems = [gsem0, gsem1]
  ssems = [ssem0, ssem1]
  woffs = [min(k * BIG, RPT - BIG) for k in range(13)]

  def zrow(r, _):
    h3[0, r, :] = jnp.zeros((F,), jnp.float32)
    return 0
  lax.fori_loop(0, BIG, zrow, 0)
  def zv(i, _):
    wv1[0, pl.ds(i * 16, 16)] = jnp.zeros((16,), jnp.float32)
    return 0
  lax.fori_loop(0, BIG // 16, zv, 0)
  zds = []
  for off in woffs:
    zds.append(pltpu.async_copy(h3.at[0], u_sh.at[pl.ds(r0 + off, BIG)],
                                ssem0))
    zds.append(pltpu.async_copy(wv1.at[0], den_sh.at[pl.ds(r0 + off, BIG)],
                                ssem0))
  for d in zds:
    d.wait()
  plsc.subcore_barrier()

  e0 = sid * (2 * BIG * NPAIR)

  def compute(p):
    def grp(g, _):
      o = pl.multiple_of(g * 16, 16)
      v = as1[p, pl.ds(o, 16)] + ad1[p, pl.ds(o, 16)]
      e = jnp.where(v >= 0.0, v, 0.2 * v)
      w = jnp.exp(e)
      wv1[p, pl.ds(o, 16)] = w
      for l in range(16):
        h3[p, o + l, :] = h3[p, o + l, :] * w[l]
      return 0
    lax.fori_loop(0, BIG // 16, grp, 0)

  def run_pair(swap, aS_t, aD_t, h_t, ia, ib):
    def sel(p):
      return (di.at[p], si.at[p]) if swap else (si.at[p], di.at[p])
    def fire_gath(p, s_i, d_i):
      return [pltpu.async_copy(aS_t.at[s_i], as1.at[p], gsems[p]),
              pltpu.async_copy(aD_t.at[d_i], ad1.at[p], gsems[p]),
              pltpu.async_copy(h_t.at[s_i], h3.at[p], gsems[p])]
    def fire_scat(p, d_i):
      return [pltpu.async_copy(h3.at[p], u_sh.at[d_i], ssems[p], add=True),
              pltpu.async_copy(wv1.at[p], den_sh.at[d_i], ssems[p],
                               add=True)]
    s0, d0 = sel(0)
    s1, d1 = sel(1)
    for d in ia:
      d.wait()
    g0 = fire_gath(0, s0, d0)
    for d in ib:
      d.wait()
    g1 = fire_gath(1, s1, d1)
    for d in g0:
      d.wait()
    compute(0)
    sc0 = fire_scat(0, d0)
    for d in g1:
      d.wait()
    compute(1)
    sc1 = fire_scat(1, d1)
    for d in sc0:
      d.wait()
    for d in sc1:
      d.wait()

  def pair(cc, _):
    base = e0 + cc * (2 * BIG)
    ia = [pltpu.async_copy(src_h.at[pl.ds(base, BIG)], si.at[0], isems[0]),
          pltpu.async_copy(dst_h.at[pl.ds(base, BIG)], di.at[0], isems[0])]
    ib = [pltpu.async_copy(src_h.at[pl.ds(base + BIG, BIG)], si.at[1],
                           isems[1]),
          pltpu.async_copy(dst_h.at[pl.ds(base + BIG, BIG)], di.at[1],
                           isems[1])]

    @pl.when(cid == 0)
    def _():
      run_pair(False, aFs_h, aFd_h, hF_h, ia, ib)

    @pl.when(cid == 1)
    def _():
      run_pair(True, aRs_h, aRd_h, hR_h, ia, ib)

    return 0

  lax.fori_loop(0, NPAIR, pair, 0)
  plsc.subcore_barrier()

  def flush(u_o, den_o):
    for k, off in enumerate(woffs):
      p = k % 2
      rd = [pltpu.async_copy(u_sh.at[pl.ds(r0 + off, BIG)], h3.at[p],
                             gsems[p]),
            pltpu.async_copy(den_sh.at[pl.ds(r0 + off, BIG)], wv1.at[p],
                             gsems[p])]
      for d in rd:
        d.wait()
      wr = [pltpu.async_copy(h3.at[p], u_o.at[pl.ds(r0 + off, BIG)],
                             ssems[p]),
            pltpu.async_copy(wv1.at[p], den_o.at[pl.ds(r0 + off, BIG)],
                             ssems[p])]
      for d in wr:
        d.wait()

  @pl.when(cid == 0)
  def _():
    flush(uF_o, denF_o)

  @pl.when(cid == 1)
  def _():
    flush(uR_o, denR_o)


@functools.partial(
    pl.kernel,
    out_type=[
        jax.ShapeDtypeStruct((N2,), jnp.float32),
        jax.ShapeDtypeStruct((N2,), jnp.float32),
        jax.ShapeDtypeStruct((N2,), jnp.float32),
        jax.ShapeDtypeStruct((N2,), jnp.float32),
    ],
    mesh=_mesh,
    compiler_params=pltpu.CompilerParams(use_tc_tiling_on_sc=False),
    scratch_types=[
        pltpu.VMEM((2, BIG2), jnp.int32),
        pltpu.VMEM((2, BIG2), jnp.int32),
        pltpu.VMEM((2, BIG2), jnp.float32),
        pltpu.VMEM((2, BIG2), jnp.float32),
        pltpu.VMEM((2, BIG2), jnp.float32),
        pltpu.VMEM((2, BIG2), jnp.float32),
        pltpu.VMEM((16,), jnp.float32),
        pltpu.SemaphoreType.DMA,
        pltpu.SemaphoreType.DMA,
        pltpu.SemaphoreType.DMA,
        pltpu.SemaphoreType.DMA,
        pltpu.SemaphoreType.DMA,
        pltpu.SemaphoreType.DMA,
        pltpu.VMEM_SHARED((N2,), jnp.float32),
        pltpu.VMEM_SHARED((N2,), jnp.float32),
        pltpu.VMEM_SHARED((N2,), jnp.float32),
    ],
)
def _sc_layer2(src_h, dst_h, TF_h, TR_h, sc_h,
               u2F_o, d2F_o, u2R_o, d2R_o,
               si, di, ts, td, wv1, mv1, scv,
               isem0, isem1, gsem0, gsem1, ssem0, ssem1,
               u_sh, den_sh, t_sp):
  cid = lax.axis_index("c")
  sid = lax.axis_index("s")
  r0 = pl.multiple_of(sid * RPT, 8)
  isems = [isem0, isem1]
  gsems = [gsem0, gsem1]
  ssems = [ssem0, ssem1]
  woffs = [min(k * BIG2, RPT - BIG2) for k in range(3)]

  pltpu.sync_copy(sc_h, scv)
  scs = scv[...]
  sa = jnp.where(cid == 0, scs[0], scs[2])
  sb = jnp.where(cid == 0, scs[1], scs[3])

  def zv(i, _):
    wv1[0, pl.ds(i * 16, 16)] = jnp.zeros((16,), jnp.float32)
    return 0
  lax.fori_loop(0, BIG2 // 16, zv, 0)
  zds = []
  for off in woffs:
    zds.append(pltpu.async_copy(wv1.at[0], u_sh.at[pl.ds(r0 + off, BIG2)],
                                ssem0))
    zds.append(pltpu.async_copy(wv1.at[0], den_sh.at[pl.ds(r0 + off, BIG2)],
                                ssem0))
  def stage_t(T_h):
    for k, off in enumerate(woffs):
      p = k % 2
      d = pltpu.async_copy(T_h.at[pl.ds(r0 + off, BIG2)], mv1.at[p],
                           gsems[p])
      d.wait()
      d = pltpu.async_copy(mv1.at[p], t_sp.at[pl.ds(r0 + off, BIG2)],
                           ssems[p])
      d.wait()

  @pl.when(cid == 0)
  def _():
    stage_t(TF_h)

  @pl.when(cid == 1)
  def _():
    stage_t(TR_h)

  for d in zds:
    d.wait()
  plsc.subcore_barrier()

  e0 = sid * (2 * BIG2 * NPAIR2)

  def compute(p):
    def grp(g, _):
      o = pl.multiple_of(g * 16, 16)
      vs = ts[p, pl.ds(o, 16)]
      v = sa * vs + sb * td[p, pl.ds(o, 16)]
      e = jnp.where(v >= 0.0, v, 0.2 * v)
      w = jnp.exp(e)
      wv1[p, pl.ds(o, 16)] = w
      mv1[p, pl.ds(o, 16)] = w * vs
      return 0
    lax.fori_loop(0, BIG2 // 16, grp, 0)

  def run_pair(swap, T_t, ia, ib):
    def sel(p):
      return (di.at[p], si.at[p]) if swap else (si.at[p], di.at[p])
    def fire_gath(p, s_i, d_i):
      return [pltpu.async_copy(t_sp.at[s_i], ts.at[p], gsems[p]),
              pltpu.async_copy(t_sp.at[d_i], td.at[p], gsems[p])]
    def fire_scat(p, d_i):
      return [pltpu.async_copy(mv1.at[p], u_sh.at[d_i], ssems[p], add=True),
              pltpu.async_copy(wv1.at[p], den_sh.at[d_i], ssems[p],
                               add=True)]
    s0, d0 = sel(0)
    s1, d1 = sel(1)
    for d in ia:
      d.wait()
    g0 = fire_gath(0, s0, d0)
    for d in ib:
      d.wait()
    g1 = fire_gath(1, s1, d1)
    for d in g0:
      d.wait()
    compute(0)
    sc0 = fire_scat(0, d0)
    for d in g1:
      d.wait()
    compute(1)
    sc1 = fire_scat(1, d1)
    for d in sc0:
      d.wait()
    for d in sc1:
      d.wait()

  def pair(cc, _):
    base = e0 + cc * (2 * BIG2)
    ia = [pltpu.async_copy(src_h.at[pl.ds(base, BIG2)], si.at[0], isems[0]),
          pltpu.async_copy(dst_h.at[pl.ds(base, BIG2)], di.at[0], isems[0])]
    ib = [pltpu.async_copy(src_h.at[pl.ds(base + BIG2, BIG2)], si.at[1],
                           isems[1]),
          pltpu.async_copy(dst_h.at[pl.ds(base + BIG2, BIG2)], di.at[1],
                           isems[1])]

    @pl.when(cid == 0)
    def _():
      run_pair(False, TF_h, ia, ib)

    @pl.when(cid == 1)
    def _():
      run_pair(True, TR_h, ia, ib)

    return 0

  lax.fori_loop(0, NPAIR2, pair, 0)
  plsc.subcore_barrier()

  def flush(u_o, den_o):
    for k, off in enumerate(woffs):
      p = k % 2
      rd = [pltpu.async_copy(u_sh.at[pl.ds(r0 + off, BIG2)], mv1.at[p],
                             gsems[p]),
            pltpu.async_copy(den_sh.at[pl.ds(r0 + off, BIG2)], wv1.at[p],
                             gsems[p])]
      for d in rd:
        d.wait()
      wr = [pltpu.async_copy(mv1.at[p], u_o.at[pl.ds(r0 + off, BIG2)],
                             ssems[p]),
            pltpu.async_copy(wv1.at[p], den_o.at[pl.ds(r0 + off, BIG2)],
                             ssems[p])]
      for d in wr:
        d.wait()

  @pl.when(cid == 0)
  def _():
    flush(u2F_o, d2F_o)

  @pl.when(cid == 1)
  def _():
    flush(u2R_o, d2R_o)


BLK = 3128


def _tca_body(x_ref, MF_ref, vasF_ref, vadF_ref, MR_ref, vasR_ref, vadR_ref,
              hF_ref, aFs_ref, aFd_ref, hR_ref, aRs_ref, aRd_ref):
  x = x_ref[...]
  hF = jnp.dot(x, MF_ref[...], preferred_element_type=jnp.float32)
  hF_ref[...] = hF
  aFs_ref[...] = jnp.dot(hF, vasF_ref[...], preferred_element_type=jnp.float32)
  aFd_ref[...] = jnp.dot(hF, vadF_ref[...], preferred_element_type=jnp.float32)
  hR = jnp.dot(x, MR_ref[...], preferred_element_type=jnp.float32)
  hR_ref[...] = hR
  aRs_ref[...] = jnp.dot(hR, vasR_ref[...], preferred_element_type=jnp.float32)
  aRd_ref[...] = jnp.dot(hR, vadR_ref[...], preferred_element_type=jnp.float32)


def _tcb_body(uF_ref, dF_ref, uR_ref, dR_ref, b1_ref, w2F_ref, b1r_ref,
              w2R_ref, TF_ref, TR_ref):
  x1F = jnp.maximum(uF_ref[...] / (dF_ref[...] + EPS) + b1_ref[...], 0.0)
  TF_ref[...] = jnp.dot(x1F, w2F_ref[...], preferred_element_type=jnp.float32)
  x1R = jnp.maximum(uR_ref[...] / (dR_ref[...] + EPS) + b1r_ref[...], 0.0)
  TR_ref[...] = jnp.dot(x1R, w2R_ref[...], preferred_element_type=jnp.float32)


def _tcc_body(u2F_ref, d2F_ref, u2R_ref, d2R_ref, bb_ref, out_ref):
  bb = bb_ref[...]
  oF = u2F_ref[...] / (d2F_ref[...] + EPS) + bb[0, 0]
  oR = u2R_ref[...] / (d2R_ref[...] + EPS) + bb[0, 1]
  out_ref[...] = (oF + oR) * 0.5


def _row_spec(cols):
  return pl.BlockSpec((BLK, cols), lambda i: (i, 0))


def _full_spec(shape):
  return pl.BlockSpec(shape, lambda i: tuple(0 for _ in shape))


def kernel(x, edge_index, W1, a1_src, a1_dst, b1, W2, a2_src, a2_dst, b2,
           W1r, a1r_src, a1r_dst, b1r, W2r, a2r_src, a2r_dst, b2r):
  pad_e = jnp.full((EP - E,), N, jnp.int32)
  src1 = jnp.concatenate([edge_index[0], pad_e])
  dst1 = jnp.concatenate([edge_index[1], pad_e])
  xp = jnp.pad(x, ((0, N2 - N), (0, 0)))

  MF = W1.T
  vasF = a1_src[:, None]
  vadF = a1_dst[:, None]
  MR = W1r.T
  vasR = a1r_src[:, None]
  vadR = a1r_dst[:, None]
  w2F = W2.T
  w2R = W2r.T
  sc16 = jnp.zeros((16,), jnp.float32).at[0].set(a2_src[0]).at[1].set(
      a2_dst[0]).at[2].set(a2r_src[0]).at[3].set(a2r_dst[0])
  bb2 = jnp.stack([b2[0], b2r[0]])[None, :]

  grid = (N2 // BLK,)
  f32 = jnp.float32

  hF, aFs, aFd, hR, aRs, aRd = pl.pallas_call(
      _tca_body,
      grid=grid,
      in_specs=[_row_spec(3), _full_spec((3, F)), _full_spec((F, 1)),
                _full_spec((F, 1)), _full_spec((3, F)), _full_spec((F, 1)),
                _full_spec((F, 1))],
      out_specs=[_row_spec(F), _row_spec(1), _row_spec(1),
                 _row_spec(F), _row_spec(1), _row_spec(1)],
      out_shape=[jax.ShapeDtypeStruct((N2, F), f32),
                 jax.ShapeDtypeStruct((N2, 1), f32),
                 jax.ShapeDtypeStruct((N2, 1), f32),
                 jax.ShapeDtypeStruct((N2, F), f32),
                 jax.ShapeDtypeStruct((N2, 1), f32),
                 jax.ShapeDtypeStruct((N2, 1), f32)],
  )(xp, MF, vasF, vadF, MR, vasR, vadR)

  uF, denF, uR, denR = _sc_layer1(
      src1, dst1, aFs.reshape(N2), aFd.reshape(N2), hF,
      aRs.reshape(N2), aRd.reshape(N2), hR)

  TF, TR = pl.pallas_call(
      _tcb_body,
      grid=grid,
      in_specs=[_row_spec(F), _row_spec(1), _row_spec(F), _row_spec(1),
                _full_spec((1, F)), _full_spec((F, 1)), _full_spec((1, F)),
                _full_spec((F, 1))],
      out_specs=[_row_spec(1)] * 2,
      out_shape=[jax.ShapeDtypeStruct((N2, 1), f32)] * 2,
  )(uF, denF.reshape(N2, 1), uR, denR.reshape(N2, 1),
    b1[None, :], w2F, b1r[None, :], w2R)

  u2F, d2F, u2R, d2R = _sc_layer2(
      src1, dst1, TF.reshape(N2), TR.reshape(N2), sc16)

  out = pl.pallas_call(
      _tcc_body,
      grid=grid,
      in_specs=[_row_spec(1), _row_spec(1), _row_spec(1), _row_spec(1),
                _full_spec((1, 2))],
      out_specs=_row_spec(1),
      out_shape=jax.ShapeDtypeStruct((N2, 1), f32),
  )(u2F.reshape(N2, 1), d2F.reshape(N2, 1), u2R.reshape(N2, 1),
    d2R.reshape(N2, 1), bb2)

  return out[:N]


# speedup vs baseline: 1.3850x; 1.0215x over previous
import functools

import jax
import jax.numpy as jnp
from jax import lax
from jax.experimental import pallas as pl
from jax.experimental.pallas import tpu as pltpu
from jax.experimental.pallas import tpu_sc as plsc

N = 100000
E = 3200000
F = 16
NTILES = 16
N2 = 100096
RPT = N2 // NTILES
ZROWS = 368
BIG = 512
NPAIR = 198
BIG2 = 3072
NPAIR2 = 33
EP = NTILES * BIG * 2 * NPAIR
EPS = 1e-16

_mesh = plsc.VectorSubcoreMesh(core_axis_name="c", subcore_axis_name="s")


def _zero_1d(ref, n):
  z = jnp.zeros((16,), jnp.float32)
  def body(i, _):
    ref[pl.ds(i * 16, 16)] = z
    return 0
  lax.fori_loop(0, n // 16, body, 0)


@functools.partial(
    pl.kernel,
    out_type=[
        jax.ShapeDtypeStruct((N2, F), jnp.float32),
        jax.ShapeDtypeStruct((N2,), jnp.float32),
        jax.ShapeDtypeStruct((N2, F), jnp.float32),
        jax.ShapeDtypeStruct((N2,), jnp.float32),
    ],
    mesh=_mesh,
    compiler_params=pltpu.CompilerParams(use_tc_tiling_on_sc=False),
    scratch_types=[
        pltpu.VMEM((4, BIG), jnp.int32),
        pltpu.VMEM((4, BIG), jnp.int32),
        pltpu.VMEM((2, BIG), jnp.float32),
        pltpu.VMEM((2, BIG), jnp.float32),
        pltpu.VMEM((2, BIG), jnp.float32),
        pltpu.VMEM((2, BIG, F), jnp.float32),
        pltpu.VMEM_SHARED((N2, F), jnp.float32),
        pltpu.VMEM_SHARED((N2,), jnp.float32),
        pltpu.SemaphoreType.DMA,
        pltpu.SemaphoreType.DMA,
        pltpu.SemaphoreType.DMA,
        pltpu.SemaphoreType.DMA,
        pltpu.SemaphoreType.DMA,
        pltpu.SemaphoreType.DMA,
        pltpu.SemaphoreType.DMA,
        pltpu.SemaphoreType.DMA,
    ],
)
def _sc_layer1(src_h, dst_h, aFs_h, aFd_h, hF_h, aRs_h, aRd_h, hR_h,
               uF_o, denF_o, uR_o, denR_o,
               si, di, as1, ad1, wv1, h3, u_sh, den_sh,
               isem0, isem1, isem2, isem3, gsem0, gsem1, ssem0, ssem1):
  cid = lax.axis_index("c")
  sid = lax.axis_index("s")
  r0 = pl.multiple_of(sid * RPT, 8)
  isems = [isem0, isem1, isem2, isem3]
  gsems = [gsem0, gsem1]
  ssems = [ssem0, ssem1]
  woffs = [min(k * BIG, RPT - BIG) for k in range(13)]

  def zrow(r, _):
    h3[0, r, :] = jnp.zeros((F,), jnp.float32)
    return 0
  lax.fori_loop(0, BIG, zrow, 0)
  def zv(i, _):
    wv1[0, pl.ds(i * 16, 16)] = jnp.zeros((16,), jnp.float32)
    return 0
  lax.fori_loop(0, BIG // 16, zv, 0)
  zds = []
  for off in woffs:
    zds.append(pltpu.async_copy(h3.at[0], u_sh.at[pl.ds(r0 + off, BIG)],
                                ssem0))
    zds.append(pltpu.async_copy(wv1.at[0], den_sh.at[pl.ds(r0 + off, BIG)],
                                ssem0))
  for d in zds:
    d.wait()
  plsc.subcore_barrier()

  e0 = sid * (2 * BIG * NPAIR)

  def compute(p):
    def grp(g, _):
      o = pl.multiple_of(g * 16, 16)
      v = as1[p, pl.ds(o, 16)] + ad1[p, pl.ds(o, 16)]
      e = jnp.where(v >= 0.0, v, 0.2 * v)
      w = jnp.exp(e)
      wv1[p, pl.ds(o, 16)] = w
      for l in range(16):
        h3[p, o + l, :] = h3[p, o + l, :] * w[l]
      return 0
    lax.fori_loop(0, BIG // 16, grp, 0)

  def fire_idx(base, sets):
    for k, q in enumerate(sets):
      pltpu.async_copy(src_h.at[pl.ds(base + k * BIG, BIG)], si.at[q],
                       isems[q])
      pltpu.async_copy(dst_h.at[pl.ds(base + k * BIG, BIG)], di.at[q],
                       isems[q])

  def drain_idx(sets):
    for q in sets:
      pltpu.make_async_copy(src_h.at[pl.ds(e0, BIG)], si.at[q],
                            isems[q]).wait()
      pltpu.make_async_copy(dst_h.at[pl.ds(e0, BIG)], di.at[q],
                            isems[q]).wait()

  def run_pair(swap, aS_t, aD_t, h_t, sets, next_sets, next_base):
    q0, q1 = sets
    def sel(q):
      return (di.at[q], si.at[q]) if swap else (si.at[q], di.at[q])
    def fire_gath(p, s_i, d_i):
      return [pltpu.async_copy(aS_t.at[s_i], as1.at[p], gsems[p]),
              pltpu.async_copy(aD_t.at[d_i], ad1.at[p], gsems[p]),
              pltpu.async_copy(h_t.at[s_i], h3.at[p], gsems[p])]
    def fire_scat(p, d_i):
      return [pltpu.async_copy(h3.at[p], u_sh.at[d_i], ssems[p], add=True),
              pltpu.async_copy(wv1.at[p], den_sh.at[d_i], ssems[p],
                               add=True)]
    s0, d0 = sel(q0)
    s1, d1 = sel(q1)
    drain_idx(sets)
    g0 = fire_gath(0, s0, d0)
    g1 = fire_gath(1, s1, d1)
    for d in g0:
      d.wait()
    compute(0)
    sc0 = fire_scat(0, d0)
    for d in g1:
      d.wait()
    fire_idx(next_base, next_sets)
    compute(1)
    sc1 = fire_scat(1, d1)
    for d in sc0:
      d.wait()
    for d in sc1:
      d.wait()

  A = (0, 1)
  B = (2, 3)

  def pair2(k, _):
    base = e0 + k * (4 * BIG)

    @pl.when(cid == 0)
    def _():
      run_pair(False, aFs_h, aFd_h, hF_h, A, B, base + 2 * BIG)
      run_pair(False, aFs_h, aFd_h, hF_h, B, A, base + 4 * BIG)

    @pl.when(cid == 1)
    def _():
      run_pair(True, aRs_h, aRd_h, hR_h, A, B, base + 2 * BIG)
      run_pair(True, aRs_h, aRd_h, hR_h, B, A, base + 4 * BIG)

    return 0

  fire_idx(e0, A)
  lax.fori_loop(0, NPAIR // 2, pair2, 0)
  drain_idx(A)
  plsc.subcore_barrier()

  def flush(u_o, den_o):
    for k, off in enumerate(woffs):
      p = k % 2
      rd = [pltpu.async_copy(u_sh.at[pl.ds(r0 + off, BIG)], h3.at[p],
                             gsems[p]),
            pltpu.async_copy(den_sh.at[pl.ds(r0 + off, BIG)], wv1.at[p],
                             gsems[p])]
      for d in rd:
        d.wait()
      wr = [pltpu.async_copy(h3.at[p], u_o.at[pl.ds(r0 + off, BIG)],
                             ssems[p]),
            pltpu.async_copy(wv1.at[p], den_o.at[pl.ds(r0 + off, BIG)],
                             ssems[p])]
      for d in wr:
        d.wait()

  @pl.when(cid == 0)
  def _():
    flush(uF_o, denF_o)

  @pl.when(cid == 1)
  def _():
    flush(uR_o, denR_o)


@functools.partial(
    pl.kernel,
    out_type=[
        jax.ShapeDtypeStruct((N2,), jnp.float32),
        jax.ShapeDtypeStruct((N2,), jnp.float32),
        jax.ShapeDtypeStruct((N2,), jnp.float32),
        jax.ShapeDtypeStruct((N2,), jnp.float32),
    ],
    mesh=_mesh,
    compiler_params=pltpu.CompilerParams(use_tc_tiling_on_sc=False),
    scratch_types=[
        pltpu.VMEM((2, BIG2), jnp.int32),
        pltpu.VMEM((2, BIG2), jnp.int32),
        pltpu.VMEM((2, BIG2), jnp.float32),
        pltpu.VMEM((2, BIG2), jnp.float32),
        pltpu.VMEM((2, BIG2), jnp.float32),
        pltpu.VMEM((2, BIG2), jnp.float32),
        pltpu.VMEM((16,), jnp.float32),
        pltpu.SemaphoreType.DMA,
        pltpu.SemaphoreType.DMA,
        pltpu.SemaphoreType.DMA,
        pltpu.SemaphoreType.DMA,
        pltpu.SemaphoreType.DMA,
        pltpu.SemaphoreType.DMA,
        pltpu.VMEM_SHARED((N2,), jnp.float32),
        pltpu.VMEM_SHARED((N2,), jnp.float32),
        pltpu.VMEM_SHARED((N2,), jnp.float32),
    ],
)
def _sc_layer2(src_h, dst_h, TF_h, TR_h, sc_h,
               u2F_o, d2F_o, u2R_o, d2R_o,
               si, di, ts, td, wv1, mv1, scv,
               isem0, isem1, gsem0, gsem1, ssem0, ssem1,
               u_sh, den_sh, t_sp):
  cid = lax.axis_index("c")
  sid = lax.axis_index("s")
  r0 = pl.multiple_of(sid * RPT, 8)
  isems = [isem0, isem1]
  gsems = [gsem0, gsem1]
  ssems = [ssem0, ssem1]
  woffs = [min(k * BIG2, RPT - BIG2) for k in range(3)]

  pltpu.sync_copy(sc_h, scv)
  scs = scv[...]
  sa = jnp.where(cid == 0, scs[0], scs[2])
  sb = jnp.where(cid == 0, scs[1], scs[3])

  def zv(i, _):
    wv1[0, pl.ds(i * 16, 16)] = jnp.zeros((16,), jnp.float32)
    return 0
  lax.fori_loop(0, BIG2 // 16, zv, 0)
  zds = []
  for off in woffs:
    zds.append(pltpu.async_copy(wv1.at[0], u_sh.at[pl.ds(r0 + off, BIG2)],
                                ssem0))
    zds.append(pltpu.async_copy(wv1.at[0], den_sh.at[pl.ds(r0 + off, BIG2)],
                                ssem0))
  def stage_t(T_h):
    for k, off in enumerate(woffs):
      p = k % 2
      d = pltpu.async_copy(T_h.at[pl.ds(r0 + off, BIG2)], mv1.at[p],
                           gsems[p])
      d.wait()
      d = pltpu.async_copy(mv1.at[p], t_sp.at[pl.ds(r0 + off, BIG2)],
                           ssems[p])
      d.wait()

  @pl.when(cid == 0)
  def _():
    stage_t(TF_h)

  @pl.when(cid == 1)
  def _():
    stage_t(TR_h)

  for d in zds:
    d.wait()
  plsc.subcore_barrier()

  e0 = sid * (2 * BIG2 * NPAIR2)

  def compute(p):
    def grp(g, _):
      o = pl.multiple_of(g * 16, 16)
      vs = ts[p, pl.ds(o, 16)]
      v = sa * vs + sb * td[p, pl.ds(o, 16)]
      e = jnp.where(v >= 0.0, v, 0.2 * v)
      w = jnp.exp(e)
      wv1[p, pl.ds(o, 16)] = w
      mv1[p, pl.ds(o, 16)] = w * vs
      return 0
    lax.fori_loop(0, BIG2 // 16, grp, 0)

  def run_pair(swap, T_t, ia, ib):
    def sel(p):
      return (di.at[p], si.at[p]) if swap else (si.at[p], di.at[p])
    def fire_gath(p, s_i, d_i):
      return [pltpu.async_copy(t_sp.at[s_i], ts.at[p], gsems[p]),
              pltpu.async_copy(t_sp.at[d_i], td.at[p], gsems[p])]
    def fire_scat(p, d_i):
      return [pltpu.async_copy(mv1.at[p], u_sh.at[d_i], ssems[p], add=True),
              pltpu.async_copy(wv1.at[p], den_sh.at[d_i], ssems[p],
                               add=True)]
    s0, d0 = sel(0)
    s1, d1 = sel(1)
    for d in ia:
      d.wait()
    g0 = fire_gath(0, s0, d0)
    for d in ib:
      d.wait()
    g1 = fire_gath(1, s1, d1)
    for d in g0:
      d.wait()
    compute(0)
    sc0 = fire_scat(0, d0)
    for d in g1:
      d.wait()
    compute(1)
    sc1 = fire_scat(1, d1)
    for d in sc0:
      d.wait()
    for d in sc1:
      d.wait()

  def pair(cc, _):
    base = e0 + cc * (2 * BIG2)
    ia = [pltpu.async_copy(src_h.at[pl.ds(base, BIG2)], si.at[0], isems[0]),
          pltpu.async_copy(dst_h.at[pl.ds(base, BIG2)], di.at[0], isems[0])]
    ib = [pltpu.async_copy(src_h.at[pl.ds(base + BIG2, BIG2)], si.at[1],
                           isems[1]),
          pltpu.async_copy(dst_h.at[pl.ds(base + BIG2, BIG2)], di.at[1],
                           isems[1])]

    @pl.when(cid == 0)
    def _():
      run_pair(False, TF_h, ia, ib)

    @pl.when(cid == 1)
    def _():
      run_pair(True, TR_h, ia, ib)

    return 0

  lax.fori_loop(0, NPAIR2, pair, 0)
  plsc.subcore_barrier()

  def flush(u_o, den_o):
    for k, off in enumerate(woffs):
      p = k % 2
      rd = [pltpu.async_copy(u_sh.at[pl.ds(r0 + off, BIG2)], mv1.at[p],
                             gsems[p]),
            pltpu.async_copy(den_sh.at[pl.ds(r0 + off, BIG2)], wv1.at[p],
                             gsems[p])]
      for d in rd:
        d.wait()
      wr = [pltpu.async_copy(mv1.at[p], u_o.at[pl.ds(r0 + off, BIG2)],
                             ssems[p]),
            pltpu.async_copy(wv1.at[p], den_o.at[pl.ds(r0 + off, BIG2)],
                             ssems[p])]
      for d in wr:
        d.wait()

  @pl.when(cid == 0)
  def _():
    flush(u2F_o, d2F_o)

  @pl.when(cid == 1)
  def _():
    flush(u2R_o, d2R_o)


BLK = 3128


def _tca_body(x_ref, MF_ref, vasF_ref, vadF_ref, MR_ref, vasR_ref, vadR_ref,
              hF_ref, aFs_ref, aFd_ref, hR_ref, aRs_ref, aRd_ref):
  x = x_ref[...]
  hF = jnp.dot(x, MF_ref[...], preferred_element_type=jnp.float32)
  hF_ref[...] = hF
  aFs_ref[...] = jnp.dot(hF, vasF_ref[...], preferred_element_type=jnp.float32)
  aFd_ref[...] = jnp.dot(hF, vadF_ref[...], preferred_element_type=jnp.float32)
  hR = jnp.dot(x, MR_ref[...], preferred_element_type=jnp.float32)
  hR_ref[...] = hR
  aRs_ref[...] = jnp.dot(hR, vasR_ref[...], preferred_element_type=jnp.float32)
  aRd_ref[...] = jnp.dot(hR, vadR_ref[...], preferred_element_type=jnp.float32)


def _tcb_body(uF_ref, dF_ref, uR_ref, dR_ref, b1_ref, w2F_ref, b1r_ref,
              w2R_ref, TF_ref, TR_ref):
  x1F = jnp.maximum(uF_ref[...] / (dF_ref[...] + EPS) + b1_ref[...], 0.0)
  TF_ref[...] = jnp.dot(x1F, w2F_ref[...], preferred_element_type=jnp.float32)
  x1R = jnp.maximum(uR_ref[...] / (dR_ref[...] + EPS) + b1r_ref[...], 0.0)
  TR_ref[...] = jnp.dot(x1R, w2R_ref[...], preferred_element_type=jnp.float32)


def _tcc_body(u2F_ref, d2F_ref, u2R_ref, d2R_ref, bb_ref, out_ref):
  bb = bb_ref[...]
  oF = u2F_ref[...] / (d2F_ref[...] + EPS) + bb[0, 0]
  oR = u2R_ref[...] / (d2R_ref[...] + EPS) + bb[0, 1]
  out_ref[...] = (oF + oR) * 0.5


def _row_spec(cols):
  return pl.BlockSpec((BLK, cols), lambda i: (i, 0))


def _full_spec(shape):
  return pl.BlockSpec(shape, lambda i: tuple(0 for _ in shape))


def kernel(x, edge_index, W1, a1_src, a1_dst, b1, W2, a2_src, a2_dst, b2,
           W1r, a1r_src, a1r_dst, b1r, W2r, a2r_src, a2r_dst, b2r):
  pad_e = jnp.full((EP - E + 2 * BIG2,), N, jnp.int32)
  src1 = jnp.concatenate([edge_index[0], pad_e])
  dst1 = jnp.concatenate([edge_index[1], pad_e])
  xp = jnp.pad(x, ((0, N2 - N), (0, 0)))

  MF = W1.T
  vasF = a1_src[:, None]
  vadF = a1_dst[:, None]
  MR = W1r.T
  vasR = a1r_src[:, None]
  vadR = a1r_dst[:, None]
  w2F = W2.T
  w2R = W2r.T
  sc16 = jnp.zeros((16,), jnp.float32).at[0].set(a2_src[0]).at[1].set(
      a2_dst[0]).at[2].set(a2r_src[0]).at[3].set(a2r_dst[0])
  bb2 = jnp.stack([b2[0], b2r[0]])[None, :]

  grid = (N2 // BLK,)
  f32 = jnp.float32

  hF, aFs, aFd, hR, aRs, aRd = pl.pallas_call(
      _tca_body,
      grid=grid,
      in_specs=[_row_spec(3), _full_spec((3, F)), _full_spec((F, 1)),
                _full_spec((F, 1)), _full_spec((3, F)), _full_spec((F, 1)),
                _full_spec((F, 1))],
      out_specs=[_row_spec(F), _row_spec(1), _row_spec(1),
                 _row_spec(F), _row_spec(1), _row_spec(1)],
      out_shape=[jax.ShapeDtypeStruct((N2, F), f32),
                 jax.ShapeDtypeStruct((N2, 1), f32),
                 jax.ShapeDtypeStruct((N2, 1), f32),
                 jax.ShapeDtypeStruct((N2, F), f32),
                 jax.ShapeDtypeStruct((N2, 1), f32),
                 jax.ShapeDtypeStruct((N2, 1), f32)],
  )(xp, MF, vasF, vadF, MR, vasR, vadR)

  uF, denF, uR, denR = _sc_layer1(
      src1, dst1, aFs.reshape(N2), aFd.reshape(N2), hF,
      aRs.reshape(N2), aRd.reshape(N2), hR)

  TF, TR = pl.pallas_call(
      _tcb_body,
      grid=grid,
      in_specs=[_row_spec(F), _row_spec(1), _row_spec(F), _row_spec(1),
                _full_spec((1, F)), _full_spec((F, 1)), _full_spec((1, F)),
                _full_spec((F, 1))],
      out_specs=[_row_spec(1)] * 2,
      out_shape=[jax.ShapeDtypeStruct((N2, 1), f32)] * 2,
  )(uF, denF.reshape(N2, 1), uR, denR.reshape(N2, 1),
    b1[None, :], w2F, b1r[None, :], w2R)

  u2F, d2F, u2R, d2R = _sc_layer2(
      src1, dst1, TF.reshape(N2), TR.reshape(N2), sc16)

  out = pl.pallas_call(
      _tcc_body,
      grid=grid,
      in_specs=[_row_spec(1), _row_spec(1), _row_spec(1), _row_spec(1),
                _full_spec((1, 2))],
      out_specs=_row_spec(1),
      out_shape=jax.ShapeDtypeStruct((N2, 1), f32),
  )(u2F.reshape(N2, 1), d2F.reshape(N2, 1), u2R.reshape(N2, 1),
    d2R.reshape(N2, 1), bb2)

  return out[:N]
